# baseline (speedup 1.0000x reference)
import jax
import jax.numpy as jnp
from jax.experimental import pallas as pl
from jax.experimental.pallas import tpu as pltpu

_N = 256
_NM = 248
_PAD = _N - _NM
_ITERS = 3
_LR = 0.1
_B1, _B2, _EPS = 0.9, 0.999, 1e-8
_UB = 8
_NBLK = _NM // _UB

_INV_2NM = 1.0 / (2.0 * _NM)
_INV_MEAN = 1.0 / (_NM * _N)


def _fold(x):
    return x[..., :128] * x[..., 128:]


def _lane_prod128(y):
    s = 64
    while s >= 1:
        y = y * pltpu.roll(y, s, axis=y.ndim - 1)
        s //= 2
    return y[..., 0:1]


def _body(aw2_ref, hlat0_ref, tau0_ref, out_ref,
          hlat_ref, h_ref, gh_ref, mg_ref, vg_ref,
          tau_ref, mt_ref, vt_ref):
    aw2 = aw2_ref[...]

    lane = jax.lax.broadcasted_iota(jnp.int32, (_NM, _N), 1)
    colmask = lane >= _PAD

    hlat_ref[...] = hlat0_ref[...]
    tau_ref[...] = tau0_ref[...]
    mg_ref[...] = jnp.zeros((_NM, _N), jnp.float32)
    vg_ref[...] = jnp.zeros((_NM, _N), jnp.float32)
    mt_ref[...] = jnp.zeros((1, _N), jnp.float32)
    vt_ref[...] = jnp.zeros((1, _N), jnp.float32)

    w_io = jax.lax.broadcasted_iota(jnp.int32, (_UB, _NM, 1), 1)
    u_io = jax.lax.broadcasted_iota(jnp.int32, (_UB, _NM, 1), 0)

    def block_fwd(u0):
        h_blk = h_ref[pl.ds(u0, _UB), :]
        f = 1.0 - aw2[None, :, :] * h_blk[:, None, :]
        y = _fold(f)
        praw = _lane_prod128(y)
        tcol = jnp.where(w_io == u_io + u0, -1.0, 1.0)
        return f, y, praw, tcol

    def grad_step(t):
        hlat = hlat_ref[...]
        h = jnp.where(colmask, jax.nn.sigmoid(hlat), 0.0)
        h_ref[...] = h

        def blk(i, carry):
            u0 = i * _UB
            f, y, praw, tcol = block_fwd(u0)
            gprod = (praw - tcol) * _INV_2NM

            def fast(_):
                contrib = ((gprod * praw) / f) * aw2[None, :, :]
                return -jnp.sum(contrib, axis=1)

            def slow(_):
                fz = f == 0.0
                zc = jnp.sum(jnp.where(fz, 1.0, 0.0), axis=2, keepdims=True)
                f_safe = jnp.where(fz, 1.0, f)
                nzprod = _lane_prod128(_fold(f_safe))
                q = jnp.where(fz, jnp.where(zc == 1.0, 1.0, 0.0),
                              jnp.where(zc == 0.0, 1.0 / f_safe, 0.0))
                contrib = ((gprod * nzprod) * q) * aw2[None, :, :]
                return -jnp.sum(contrib, axis=1)

            has_zero = jnp.min(jnp.abs(y)) == 0.0
            gh_ref[pl.ds(u0, _UB), :] = jax.lax.cond(has_zero, slow, fast, 0)
            return carry

        jax.lax.fori_loop(0, _NBLK, blk, 0, unroll=2)

        tau = tau_ref[...]
        tau_col = jnp.transpose(tau)[: _NM, :]
        d = tau_col - tau + 0.1
        r = jnp.maximum(d, 0.0)
        h = h_ref[...]
        gh = gh_ref[...] + (r * r) * _INV_MEAN
        ghlat = gh * (h * (1.0 - h))

        w_mat = h * (2.0 * r)
        rowsum = jnp.sum(w_mat, axis=1, keepdims=True)
        rowsum_full = jnp.concatenate(
            [rowsum, jnp.zeros((_PAD, 1), jnp.float32)], axis=0)
        gtau = (jnp.transpose(rowsum_full)
                - jnp.sum(w_mat, axis=0, keepdims=True)) * _INV_MEAN

        c1 = 1.0 - _B1 ** t
        c2 = 1.0 - _B2 ** t
        mg = _B1 * mg_ref[...] + (1.0 - _B1) * ghlat
        vg = _B2 * vg_ref[...] + (1.0 - _B2) * (ghlat * ghlat)
        mg_ref[...] = mg
        vg_ref[...] = vg
        hlat_ref[...] = hlat - _LR * (mg / c1) / (jnp.sqrt(vg / c2) + _EPS)

        mt = _B1 * mt_ref[...] + (1.0 - _B1) * gtau
        vt = _B2 * vt_ref[...] + (1.0 - _B2) * (gtau * gtau)
        mt_ref[...] = mt
        vt_ref[...] = vt
        tau_ref[...] = tau - _LR * (mt / c1) / (jnp.sqrt(vt / c2) + _EPS)

    for t in range(1, _ITERS + 1):
        grad_step(t)

    hlat = hlat_ref[...]
    h = jnp.where(colmask, jax.nn.sigmoid(hlat), 0.0)
    h_ref[...] = h

    def loss_blk(i, acc):
        u0 = i * _UB
        _, _, praw, tcol = block_fwd(u0)
        return acc + jnp.sum((praw - tcol) ** 2)

    odd_sum = jax.lax.fori_loop(0, _NBLK, loss_blk, jnp.float32(0.0),
                                unroll=2)
    loss_odd = odd_sum / (4.0 * _NM)

    tau = tau_ref[...]
    tau_col = jnp.transpose(tau)[: _NM, :]
    r = jnp.maximum(tau_col - tau + 0.1, 0.0)
    loss_order = jnp.sum(h * (r * r)) * _INV_MEAN
    out_ref[0, 0] = loss_odd + loss_order


def kernel(A, tau_init, G_latent_init):
    aw2 = 2.0 * A[: _NM, :]
    hlat0 = jnp.pad(G_latent_init, ((0, 0), (_PAD, 0)))
    tau0 = tau_init.reshape(1, _N)

    out = pl.pallas_call(
        _body,
        out_shape=jax.ShapeDtypeStruct((1, 1), jnp.float32),
        out_specs=pl.BlockSpec(memory_space=pltpu.SMEM),
        scratch_shapes=[
            pltpu.VMEM((_NM, _N), jnp.float32),
            pltpu.VMEM((_NM, _N), jnp.float32),
            pltpu.VMEM((_NM, _N), jnp.float32),
            pltpu.VMEM((_NM, _N), jnp.float32),
            pltpu.VMEM((_NM, _N), jnp.float32),
            pltpu.VMEM((1, _N), jnp.float32),
            pltpu.VMEM((1, _N), jnp.float32),
            pltpu.VMEM((1, _N), jnp.float32),
        ],
    )(aw2, hlat0, tau0)
    return out[0, 0]


# speedup vs baseline: 1.0321x; 1.0321x over previous
import jax
import jax.numpy as jnp
from jax.experimental import pallas as pl
from jax.experimental.pallas import tpu as pltpu

_N = 256
_NM = 248
_PAD = _N - _NM
_ITERS = 3
_LR = 0.1
_B1, _B2, _EPS = 0.9, 0.999, 1e-8
_UB = 8
_NBLK = _NM // _UB

_INV_2NM = 1.0 / (2.0 * _NM)
_INV_MEAN = 1.0 / (_NM * _N)


def _fold(x):
    return x[..., :128] * x[..., 128:]


def _lane_prod128(y):
    s = 64
    while s >= 1:
        y = y * pltpu.roll(y, s, axis=y.ndim - 1)
        s //= 2
    return y[..., 0:1]


def _body(aw2_ref, hlat0_ref, tau0_ref, out_ref,
          hlat_ref, h_ref, gh_ref, mg_ref, vg_ref,
          tau_ref, mt_ref, vt_ref):
    aw2 = aw2_ref[...]

    lane = jax.lax.broadcasted_iota(jnp.int32, (_NM, _N), 1)
    colmask = lane >= _PAD

    hlat_ref[...] = hlat0_ref[...]
    tau_ref[...] = tau0_ref[...]
    mg_ref[...] = jnp.zeros((_NM, _N), jnp.float32)
    vg_ref[...] = jnp.zeros((_NM, _N), jnp.float32)
    mt_ref[...] = jnp.zeros((1, _N), jnp.float32)
    vt_ref[...] = jnp.zeros((1, _N), jnp.float32)

    w_io = jax.lax.broadcasted_iota(jnp.int32, (_UB, _NM, 1), 1)
    u_io = jax.lax.broadcasted_iota(jnp.int32, (_UB, _NM, 1), 0)

    def block_fwd(u0):
        h_blk = h_ref[pl.ds(u0, _UB), :]
        f = 1.0 - aw2[None, :, :] * h_blk[:, None, :]
        y = _fold(f)
        praw = _lane_prod128(y)
        tcol = jnp.where(w_io == u_io + u0, -1.0, 1.0)
        return f, y, praw, tcol

    def grad_step(t):
        hlat = hlat_ref[...]
        h = jnp.where(colmask, jax.nn.sigmoid(hlat), 0.0)
        h_ref[...] = h

        def blk(i, carry):
            u0 = i * _UB
            f, y, praw, tcol = block_fwd(u0)
            gprod = (praw - tcol) * _INV_2NM

            def fast(_):
                ry = (gprod * praw) / y
                c_lo = (ry * f[..., 128:]) * aw2[None, :, :128]
                c_hi = (ry * f[..., :128]) * aw2[None, :, 128:]
                return jnp.concatenate(
                    [-jnp.sum(c_lo, axis=1), -jnp.sum(c_hi, axis=1)],
                    axis=-1)

            def slow(_):
                fz = f == 0.0
                zc = jnp.sum(jnp.where(fz, 1.0, 0.0), axis=2, keepdims=True)
                f_safe = jnp.where(fz, 1.0, f)
                nzprod = _lane_prod128(_fold(f_safe))
                q = jnp.where(fz, jnp.where(zc == 1.0, 1.0, 0.0),
                              jnp.where(zc == 0.0, 1.0 / f_safe, 0.0))
                contrib = ((gprod * nzprod) * q) * aw2[None, :, :]
                return -jnp.sum(contrib, axis=1)

            has_zero = jnp.min(jnp.abs(y)) == 0.0
            gh_ref[pl.ds(u0, _UB), :] = jax.lax.cond(has_zero, slow, fast, 0)
            return carry

        jax.lax.fori_loop(0, _NBLK, blk, 0, unroll=False)

        tau = tau_ref[...]
        tau_col = jnp.transpose(tau)[: _NM, :]
        d = tau_col - tau + 0.1
        r = jnp.maximum(d, 0.0)
        h = h_ref[...]
        gh = gh_ref[...] + (r * r) * _INV_MEAN
        ghlat = gh * (h * (1.0 - h))

        w_mat = h * (2.0 * r)
        rowsum = jnp.sum(w_mat, axis=1, keepdims=True)
        rowsum_full = jnp.concatenate(
            [rowsum, jnp.zeros((_PAD, 1), jnp.float32)], axis=0)
        gtau = (jnp.transpose(rowsum_full)
                - jnp.sum(w_mat, axis=0, keepdims=True)) * _INV_MEAN

        c1 = 1.0 - _B1 ** t
        c2 = 1.0 - _B2 ** t
        mg = _B1 * mg_ref[...] + (1.0 - _B1) * ghlat
        vg = _B2 * vg_ref[...] + (1.0 - _B2) * (ghlat * ghlat)
        mg_ref[...] = mg
        vg_ref[...] = vg
        hlat_ref[...] = hlat - _LR * (mg / c1) / (jnp.sqrt(vg / c2) + _EPS)

        mt = _B1 * mt_ref[...] + (1.0 - _B1) * gtau
        vt = _B2 * vt_ref[...] + (1.0 - _B2) * (gtau * gtau)
        mt_ref[...] = mt
        vt_ref[...] = vt
        tau_ref[...] = tau - _LR * (mt / c1) / (jnp.sqrt(vt / c2) + _EPS)

    for t in range(1, _ITERS + 1):
        grad_step(t)

    hlat = hlat_ref[...]
    h = jnp.where(colmask, jax.nn.sigmoid(hlat), 0.0)
    h_ref[...] = h

    def loss_blk(i, acc):
        u0 = i * _UB
        _, _, praw, tcol = block_fwd(u0)
        return acc + jnp.sum((praw - tcol) ** 2)

    odd_sum = jax.lax.fori_loop(0, _NBLK, loss_blk, jnp.float32(0.0),
                                unroll=False)
    loss_odd = odd_sum / (4.0 * _NM)

    tau = tau_ref[...]
    tau_col = jnp.transpose(tau)[: _NM, :]
    r = jnp.maximum(tau_col - tau + 0.1, 0.0)
    loss_order = jnp.sum(h * (r * r)) * _INV_MEAN
    out_ref[0, 0] = loss_odd + loss_order


def kernel(A, tau_init, G_latent_init):
    aw2 = 2.0 * A[: _NM, :]
    hlat0 = jnp.pad(G_latent_init, ((0, 0), (_PAD, 0)))
    tau0 = tau_init.reshape(1, _N)

    out = pl.pallas_call(
        _body,
        out_shape=jax.ShapeDtypeStruct((1, 1), jnp.float32),
        out_specs=pl.BlockSpec(memory_space=pltpu.SMEM),
        scratch_shapes=[
            pltpu.VMEM((_NM, _N), jnp.float32),
            pltpu.VMEM((_NM, _N), jnp.float32),
            pltpu.VMEM((_NM, _N), jnp.float32),
            pltpu.VMEM((_NM, _N), jnp.float32),
            pltpu.VMEM((_NM, _N), jnp.float32),
            pltpu.VMEM((1, _N), jnp.float32),
            pltpu.VMEM((1, _N), jnp.float32),
            pltpu.VMEM((1, _N), jnp.float32),
        ],
    )(aw2, hlat0, tau0)
    return out[0, 0]


# speedup vs baseline: 1.0346x; 1.0024x over previous
import jax
import jax.numpy as jnp
from jax.experimental import pallas as pl
from jax.experimental.pallas import tpu as pltpu

_N = 256
_NM = 248
_PAD = _N - _NM
_ITERS = 3
_LR = 0.1
_B1, _B2, _EPS = 0.9, 0.999, 1e-8
_UB = 8
_NBLK = _NM // _UB

_INV_2NM = 1.0 / (2.0 * _NM)
_INV_MEAN = 1.0 / (_NM * _N)


def _fold(x):
    return x[..., :128] * x[..., 128:]


def _lane_prod128(y):
    s = 64
    while s >= 1:
        y = y * pltpu.roll(y, s, axis=y.ndim - 1)
        s //= 2
    return y[..., 0:1]


def _body(aw2_ref, hlat0_ref, tau0_ref, out_ref,
          hlat_ref, h_ref, gh_ref, mg_ref, vg_ref,
          tau_ref, mt_ref, vt_ref):
    aw2 = aw2_ref[...]

    lane = jax.lax.broadcasted_iota(jnp.int32, (_NM, _N), 1)
    colmask = lane >= _PAD

    hlat_ref[...] = hlat0_ref[...]
    tau_ref[...] = tau0_ref[...]
    mg_ref[...] = jnp.zeros((_NM, _N), jnp.float32)
    vg_ref[...] = jnp.zeros((_NM, _N), jnp.float32)
    mt_ref[...] = jnp.zeros((1, _N), jnp.float32)
    vt_ref[...] = jnp.zeros((1, _N), jnp.float32)

    w_io = jax.lax.broadcasted_iota(jnp.int32, (_UB, _NM, 1), 1)
    u_io = jax.lax.broadcasted_iota(jnp.int32, (_UB, _NM, 1), 0)

    def block_fwd(u0):
        h_blk = h_ref[pl.ds(u0, _UB), :]
        f = 1.0 - aw2[None, :, :] * h_blk[:, None, :]
        y = _fold(f)
        praw = _lane_prod128(y)
        tcol = jnp.where(w_io == u_io + u0, -1.0, 1.0)
        return f, y, praw, tcol

    def grad_step(t):
        hlat = hlat_ref[...]
        h = jnp.where(colmask, jax.nn.sigmoid(hlat), 0.0)
        h_ref[...] = h

        def blk(i, carry):
            u0 = i * _UB
            f, y, praw, tcol = block_fwd(u0)
            gprod = (praw - tcol) * _INV_2NM

            def fast(_):
                contrib = ((gprod * praw) / f) * aw2[None, :, :]
                return -jnp.sum(contrib, axis=1)

            def slow(_):
                fz = f == 0.0
                zc = jnp.sum(jnp.where(fz, 1.0, 0.0), axis=2, keepdims=True)
                f_safe = jnp.where(fz, 1.0, f)
                nzprod = _lane_prod128(_fold(f_safe))
                q = jnp.where(fz, jnp.where(zc == 1.0, 1.0, 0.0),
                              jnp.where(zc == 0.0, 1.0 / f_safe, 0.0))
                contrib = ((gprod * nzprod) * q) * aw2[None, :, :]
                return -jnp.sum(contrib, axis=1)

            has_zero = jnp.min(jnp.abs(y)) == 0.0
            gh_ref[pl.ds(u0, _UB), :] = jax.lax.cond(has_zero, slow, fast, 0)
            return carry

        jax.lax.fori_loop(0, _NBLK, blk, 0, unroll=False)

        tau = tau_ref[...]
        tau_col = jnp.transpose(tau)[: _NM, :]
        d = tau_col - tau + 0.1
        r = jnp.maximum(d, 0.0)
        h = h_ref[...]
        gh = gh_ref[...] + (r * r) * _INV_MEAN
        ghlat = gh * (h * (1.0 - h))

        w_mat = h * (2.0 * r)
        rowsum = jnp.sum(w_mat, axis=1, keepdims=True)
        rowsum_full = jnp.concatenate(
            [rowsum, jnp.zeros((_PAD, 1), jnp.float32)], axis=0)
        gtau = (jnp.transpose(rowsum_full)
                - jnp.sum(w_mat, axis=0, keepdims=True)) * _INV_MEAN

        c1 = 1.0 - _B1 ** t
        c2 = 1.0 - _B2 ** t
        mg = _B1 * mg_ref[...] + (1.0 - _B1) * ghlat
        vg = _B2 * vg_ref[...] + (1.0 - _B2) * (ghlat * ghlat)
        mg_ref[...] = mg
        vg_ref[...] = vg
        hlat_ref[...] = hlat - _LR * (mg / c1) / (jnp.sqrt(vg / c2) + _EPS)

        mt = _B1 * mt_ref[...] + (1.0 - _B1) * gtau
        vt = _B2 * vt_ref[...] + (1.0 - _B2) * (gtau * gtau)
        mt_ref[...] = mt
        vt_ref[...] = vt
        tau_ref[...] = tau - _LR * (mt / c1) / (jnp.sqrt(vt / c2) + _EPS)

    for t in range(1, _ITERS + 1):
        grad_step(t)

    hlat = hlat_ref[...]
    h = jnp.where(colmask, jax.nn.sigmoid(hlat), 0.0)
    h_ref[...] = h

    def loss_blk(i, acc):
        u0 = i * _UB
        _, _, praw, tcol = block_fwd(u0)
        return acc + jnp.sum((praw - tcol) ** 2)

    odd_sum = jax.lax.fori_loop(0, _NBLK, loss_blk, jnp.float32(0.0),
                                unroll=False)
    loss_odd = odd_sum / (4.0 * _NM)

    tau = tau_ref[...]
    tau_col = jnp.transpose(tau)[: _NM, :]
    r = jnp.maximum(tau_col - tau + 0.1, 0.0)
    loss_order = jnp.sum(h * (r * r)) * _INV_MEAN
    out_ref[0, 0] = loss_odd + loss_order


def kernel(A, tau_init, G_latent_init):
    aw2 = 2.0 * A[: _NM, :]
    hlat0 = jnp.pad(G_latent_init, ((0, 0), (_PAD, 0)))
    tau0 = tau_init.reshape(1, _N)

    out = pl.pallas_call(
        _body,
        out_shape=jax.ShapeDtypeStruct((1, 1), jnp.float32),
        out_specs=pl.BlockSpec(memory_space=pltpu.SMEM),
        scratch_shapes=[
            pltpu.VMEM((_NM, _N), jnp.float32),
            pltpu.VMEM((_NM, _N), jnp.float32),
            pltpu.VMEM((_NM, _N), jnp.float32),
            pltpu.VMEM((_NM, _N), jnp.float32),
            pltpu.VMEM((_NM, _N), jnp.float32),
            pltpu.VMEM((1, _N), jnp.float32),
            pltpu.VMEM((1, _N), jnp.float32),
            pltpu.VMEM((1, _N), jnp.float32),
        ],
    )(aw2, hlat0, tau0)
    return out[0, 0]
